# baseline (speedup 1.0000x reference)
import functools

import jax
import jax.numpy as jnp
from jax import lax
from jax.experimental import pallas as pl
from jax.experimental.pallas import tpu as pltpu
from jax.experimental.pallas import tpu_sc as plsc

_BATCH = 4096
_SEQ = 200
_DIM = 32
_VOCAB = 1000

_NC = 2
_NS = 16
_NW = _NC * _NS

_BT = _BATCH // 128
_ST = _SEQ // 8
_DT = _DIM // 8
_PAD = 40

assert _BT == _NW


def _body(xt_hbm, tab_hbm, pos_hbm, out_hbm, idx_v, stage_v, out_v, pos_v,
          semi, semg, semo):
  w = lax.axis_index("s") * _NC + lax.axis_index("c")
  lane = lax.iota(jnp.int32, 16)

  pltpu.sync_copy(pos_hbm, pos_v)

  def fire_gathers(buf, wait=False):
    for sr in range(8):
      cp = (tab_hbm.at[idx_v.at[buf, sr]],
            stage_v.at[buf, pl.ds(sr * 128, 128)], semg)
      if wait:
        pltpu.make_async_copy(*cp).wait()
      else:
        pltpu.async_copy(*cp)

  pltpu.sync_copy(xt_hbm.at[0, w], idx_v.at[0])
  fire_gathers(0)
  pltpu.async_copy(xt_hbm.at[1, w], idx_v.at[1], semi)

  def st_body(st, carry):
    cur = lax.rem(st, 2)
    nxt = 1 - cur
    st1 = jnp.minimum(st + 1, _ST - 1)
    st2 = jnp.minimum(st + 2, _ST - 1)

    fire_gathers(cur, wait=True)
    pltpu.make_async_copy(
        xt_hbm.at[st1, w], idx_v.at[nxt], semi).wait()
    pltpu.async_copy(xt_hbm.at[st2, w], idx_v.at[cur], semi)
    fire_gathers(nxt)

    @pl.when(st > 0)
    def _():
      pltpu.make_async_copy(
          out_v, out_hbm.at[pl.ds(8 * (st - 1), 8), :, w], semo).wait()

    @plsc.parallel_loop(0, _DT)
    def dt_body(dt):
      d_lo = 8 * dt
      splat_lo = jnp.full((16,), 0, jnp.int32) + lax.rem(d_lo, 16)
      cols = [jnp.full((16,), 0, jnp.int32) + (d_lo + dr) for dr in range(8)]
      for sr in range(8):
        s = 8 * st + sr
        pr0 = pos_v[s, pl.ds(0, 16)]
        pr1 = pos_v[s, pl.ds(16, 16)]
        pr = jnp.where(dt < 2, pr0, pr1)
        pvs = [jnp.take_along_axis(pr, splat_lo + dr, axis=0)
               for dr in range(8)]
        for j in range(8):
          row_ix = lane + (sr * 128 + 16 * j)
          for dr in range(8):
            val = plsc.load_gather(stage_v.at[cur], [row_ix, cols[dr]])
            out_v[sr, dt, dr, pl.ds(16 * j, 16)] = val + pvs[dr]

    pltpu.async_copy(out_v, out_hbm.at[pl.ds(8 * st, 8), :, w], semo)
    return carry

  lax.fori_loop(0, _ST, st_body, 0)

  pltpu.make_async_copy(
      out_v, out_hbm.at[pl.ds(8 * (_ST - 1), 8), :, w], semo).wait()
  pltpu.make_async_copy(
      xt_hbm.at[_ST - 1, w], idx_v.at[_ST % 2], semi).wait()
  fire_gathers(_ST % 2, wait=True)


@jax.jit
def _embed(xt, nucleo_emb, pos_emb):
  mesh = plsc.VectorSubcoreMesh(
      core_axis_name="c", subcore_axis_name="s", num_cores=_NC,
      num_subcores=_NS)
  return pl.kernel(
      _body,
      out_type=jax.ShapeDtypeStruct((_SEQ, _DT, _BT, 8, 128), jnp.float32),
      mesh=mesh,
      compiler_params=pltpu.CompilerParams(
          use_tc_tiling_on_sc=False, needs_layout_passes=False),
      scratch_types=[
          pltpu.VMEM((2, 8, 128), jnp.int32),
          pltpu.VMEM((2, 8 * 128, _PAD), jnp.float32),
          pltpu.VMEM((8, _DT, 8, 128), jnp.float32),
          pltpu.VMEM((_SEQ, _DIM), jnp.float32),
          pltpu.SemaphoreType.DMA,
          pltpu.SemaphoreType.DMA,
          pltpu.SemaphoreType.DMA,
      ],
  )(xt, nucleo_emb, pos_emb)


def kernel(X, nucleo_emb, pos_emb):
  xt = X.reshape(_BT, 128, _ST, 8).transpose(2, 0, 3, 1)
  tab = jnp.pad(nucleo_emb, ((0, 0), (0, _PAD - _DIM)))
  t = _embed(xt, tab, pos_emb)
  return t.transpose(2, 4, 0, 1, 3).reshape(_BATCH, _SEQ, _DIM)


# speedup vs baseline: 1.0956x; 1.0956x over previous
import functools

import jax
import jax.numpy as jnp
from jax import lax
from jax.experimental import pallas as pl
from jax.experimental.pallas import tpu as pltpu
from jax.experimental.pallas import tpu_sc as plsc

_BATCH = 4096
_SEQ = 200
_DIM = 32
_VOCAB = 1000

_NC = 2
_NS = 16
_NW = _NC * _NS

_BT = _BATCH // 128
_ST = _SEQ // 8
_DT = _DIM // 8
_PAD = 40

assert _BT == _NW


def _body(xt_hbm, tab_hbm, pos_hbm, out_hbm, idx_v, stage_v, out_v, pos_v,
          semi, semg, semo):
  w = lax.axis_index("s") * _NC + lax.axis_index("c")
  lane = lax.iota(jnp.int32, 16)

  pltpu.sync_copy(pos_hbm, pos_v)

  def fire_gathers(buf, wait=False):
    for sr in range(8):
      cp = (tab_hbm.at[idx_v.at[buf, sr]],
            stage_v.at[buf, pl.ds(sr * 128, 128)], semg)
      if wait:
        pltpu.make_async_copy(*cp).wait()
      else:
        pltpu.async_copy(*cp)

  pltpu.sync_copy(xt_hbm.at[0, w], idx_v.at[0])
  fire_gathers(0)
  pltpu.async_copy(xt_hbm.at[1, w], idx_v.at[1], semi)

  def st_body(st, carry):
    cur = lax.rem(st, 2)
    nxt = 1 - cur
    st1 = jnp.minimum(st + 1, _ST - 1)
    st2 = jnp.minimum(st + 2, _ST - 1)

    fire_gathers(cur, wait=True)
    pltpu.make_async_copy(
        xt_hbm.at[st1, w], idx_v.at[nxt], semi).wait()
    pltpu.async_copy(xt_hbm.at[st2, w], idx_v.at[cur], semi)
    fire_gathers(nxt)

    @pl.when(st > 0)
    def _():
      pltpu.make_async_copy(
          out_v, out_hbm.at[pl.ds(8 * (st - 1), 8), :, w], semo).wait()

    @plsc.parallel_loop(0, _DT)
    def dt_body(dt):
      d_lo = 8 * dt
      splat_lo = jnp.full((16,), 0, jnp.int32) + lax.rem(d_lo, 16)
      for sr in range(8):
        s = 8 * st + sr
        pr0 = pos_v[s, pl.ds(0, 16)]
        pr1 = pos_v[s, pl.ds(16, 16)]
        pr = jnp.where(dt < 2, pr0, pr1)
        pvs = [jnp.take_along_axis(pr, splat_lo + dr, axis=0)
               for dr in range(8)]
        for j in range(8):
          base = _PAD * (sr * 128 + 16 * j) + d_lo
          for dr in range(8):
            cix = jnp.full((16,), 0, jnp.int32) + (base + dr)
            val = plsc.load_gather(stage_v.at[cur], [lane, cix])
            out_v[sr, dt, dr, pl.ds(16 * j, 16)] = val + pvs[dr]

    pltpu.async_copy(out_v, out_hbm.at[pl.ds(8 * st, 8), :, w], semo)
    return carry

  lax.fori_loop(0, _ST, st_body, 0)

  pltpu.make_async_copy(
      out_v, out_hbm.at[pl.ds(8 * (_ST - 1), 8), :, w], semo).wait()
  pltpu.make_async_copy(
      xt_hbm.at[_ST - 1, w], idx_v.at[_ST % 2], semi).wait()
  fire_gathers(_ST % 2, wait=True)


@jax.jit
def _embed(xt, nucleo_emb, pos_emb):
  mesh = plsc.VectorSubcoreMesh(
      core_axis_name="c", subcore_axis_name="s", num_cores=_NC,
      num_subcores=_NS)
  return pl.kernel(
      _body,
      out_type=jax.ShapeDtypeStruct((_SEQ, _DT, _BT, 8, 128), jnp.float32),
      mesh=mesh,
      compiler_params=pltpu.CompilerParams(
          use_tc_tiling_on_sc=False, needs_layout_passes=False),
      scratch_types=[
          pltpu.VMEM((2, 8, 128), jnp.int32),
          pltpu.VMEM((2, 8 * 128, _PAD), jnp.float32),
          pltpu.VMEM((8, _DT, 8, 128), jnp.float32),
          pltpu.VMEM((_SEQ, _DIM), jnp.float32),
          pltpu.SemaphoreType.DMA,
          pltpu.SemaphoreType.DMA,
          pltpu.SemaphoreType.DMA,
      ],
  )(xt, nucleo_emb, pos_emb)


def kernel(X, nucleo_emb, pos_emb):
  xt = X.reshape(_BT, 128, _ST, 8).transpose(2, 0, 3, 1)
  tab = jnp.pad(nucleo_emb, ((0, 0), (0, _PAD - _DIM)))
  t = _embed(xt, tab, pos_emb)
  return t.transpose(2, 4, 0, 1, 3).reshape(_BATCH, _SEQ, _DIM)


# speedup vs baseline: 2.1860x; 1.9952x over previous
import functools

import jax
import jax.numpy as jnp
from jax import lax
from jax.experimental import pallas as pl
from jax.experimental.pallas import tpu as pltpu
from jax.experimental.pallas import tpu_sc as plsc

_BATCH = 4096
_SEQ = 200
_DIM = 32
_VOCAB = 1000

_NC = 2
_NS = 16
_NW = _NC * _NS

_BT = _BATCH // 128
_ST = _SEQ // 8
_DT = _DIM // 8
_PAD = 40

assert _BT == _NW


def _body(xt_hbm, tab_hbm, pos_hbm, out_hbm, idx_v, stage_v, out_v, pos_v,
          semi, semg, semo):
  w = lax.axis_index("s") * _NC + lax.axis_index("c")
  lane = lax.iota(jnp.int32, 16)

  pltpu.sync_copy(pos_hbm, pos_v)

  def fire_gathers(buf, wait=False):
    for sr in range(8):
      cp = (tab_hbm.at[idx_v.at[buf, sr]],
            stage_v.at[buf, pl.ds(sr * 128, 128)], semg)
      if wait:
        pltpu.make_async_copy(*cp).wait()
      else:
        pltpu.async_copy(*cp)

  pltpu.sync_copy(xt_hbm.at[0, w], idx_v.at[0])
  fire_gathers(0)
  pltpu.async_copy(xt_hbm.at[1, w], idx_v.at[1], semi)

  def st_body(st, carry):
    cur = lax.rem(st, 2)
    nxt = 1 - cur
    st1 = jnp.minimum(st + 1, _ST - 1)
    st2 = jnp.minimum(st + 2, _ST - 1)

    fire_gathers(cur, wait=True)
    pltpu.make_async_copy(
        xt_hbm.at[st1, w], idx_v.at[nxt], semi).wait()
    pltpu.async_copy(xt_hbm.at[st2, w], idx_v.at[cur], semi)
    fire_gathers(nxt)

    @pl.when(st > 0)
    def _():
      pltpu.make_async_copy(
          out_v, out_hbm.at[pl.ds(8 * (st - 1), 8), :, w], semo).wait()

    @plsc.parallel_loop(0, _DT * 64, unroll=2)
    def u_body(u):
      j = lax.rem(u, 8)
      sr = lax.rem(u // 8, 8)
      dt = u // 64
      d_lo = 8 * dt
      s = 8 * st + sr
      pr0 = pos_v[s, pl.ds(0, 16)]
      pr1 = pos_v[s, pl.ds(16, 16)]
      pr = jnp.where(dt < 2, pr0, pr1)
      splat_lo = jnp.full((16,), 0, jnp.int32) + lax.rem(d_lo, 16)
      base = _PAD * (sr * 128 + 16 * j) + d_lo
      col = 16 * j
      for dr in range(8):
        cix = jnp.full((16,), 0, jnp.int32) + (base + dr)
        val = plsc.load_gather(stage_v.at[cur], [lane, cix])
        pv = jnp.take_along_axis(pr, splat_lo + dr, axis=0)
        out_v[sr, dt, dr, pl.ds(col, 16)] = val + pv

    pltpu.async_copy(out_v, out_hbm.at[pl.ds(8 * st, 8), :, w], semo)
    return carry

  lax.fori_loop(0, _ST, st_body, 0)

  pltpu.make_async_copy(
      out_v, out_hbm.at[pl.ds(8 * (_ST - 1), 8), :, w], semo).wait()
  pltpu.make_async_copy(
      xt_hbm.at[_ST - 1, w], idx_v.at[_ST % 2], semi).wait()
  fire_gathers(_ST % 2, wait=True)


@jax.jit
def _embed(xt, nucleo_emb, pos_emb):
  mesh = plsc.VectorSubcoreMesh(
      core_axis_name="c", subcore_axis_name="s", num_cores=_NC,
      num_subcores=_NS)
  return pl.kernel(
      _body,
      out_type=jax.ShapeDtypeStruct((_SEQ, _DT, _BT, 8, 128), jnp.float32),
      mesh=mesh,
      compiler_params=pltpu.CompilerParams(
          use_tc_tiling_on_sc=False, needs_layout_passes=False),
      scratch_types=[
          pltpu.VMEM((2, 8, 128), jnp.int32),
          pltpu.VMEM((2, 8 * 128, _PAD), jnp.float32),
          pltpu.VMEM((8, _DT, 8, 128), jnp.float32),
          pltpu.VMEM((_SEQ, _DIM), jnp.float32),
          pltpu.SemaphoreType.DMA,
          pltpu.SemaphoreType.DMA,
          pltpu.SemaphoreType.DMA,
      ],
  )(xt, nucleo_emb, pos_emb)


def kernel(X, nucleo_emb, pos_emb):
  xt = X.reshape(_BT, 128, _ST, 8).transpose(2, 0, 3, 1)
  tab = jnp.pad(nucleo_emb, ((0, 0), (0, _PAD - _DIM)))
  t = _embed(xt, tab, pos_emb)
  return t.transpose(2, 4, 0, 1, 3).reshape(_BATCH, _SEQ, _DIM)
